# baseline (device time: 105116 ns/iter reference)
import jax
import jax.numpy as jnp
from jax import lax
from jax.experimental import pallas as pl
from jax.experimental.pallas import tpu as pltpu

N_DEV = 8
SQ = 2048
D_MODEL = 1024
H_LOC = 8
DH = 128
NG = 4
NB = SQ // 64
JPG = NB // NG
GROUP = JPG * 64
CHUNK = SQ // N_DEV
SCALE = 0.08838834764831843


HD_INST = (
    (0, 768, (1, 3, 4)),
    (768, 768, (3, 4, 1)),
    (1536, 512, (4, 1, 3)),
)


def _tbit(my, m):
    p0 = lax.rem(my, 2)
    p1 = lax.rem(my // 2, 2)
    p2 = my // 4
    if m == 1:
        return lax.rem(p0 + p1, 2)
    if m == 3:
        return p1
    return p2


def _body(x_ref, wq_ref, k_ref, v_ref, wo_ref, out_ref,
          xp, kp, vp, qp, kvbuf, *comm):
    bufs = [[comm[i * 3 + r] for r in range(3)] for i in range(3)]
    rs_send, rs_recv, ag_send, ag_recv = comm[9:13]
    k_sem, v_sem = comm[13], comm[14]
    my = lax.axis_index("i")

    k_dma = pltpu.make_async_copy(
        k_ref.at[0, :, pl.ds(my * H_LOC, H_LOC), :], kvbuf, k_sem
    )
    k_dma.start()

    for g in range(NG):
        for j in range(JPG):
            b = g + NG * j
            dst = slice(g * GROUP + j * 64, g * GROUP + (j + 1) * 64)
            xp[dst, :] = x_ref[b * 64:(b + 1) * 64, :]

    k_dma.wait()
    for g in range(NG):
        for j in range(JPG):
            b = g + NG * j
            dst = slice(g * GROUP + j * 64, g * GROUP + (j + 1) * 64)
            kp[dst, :, :] = kvbuf[b * 64:(b + 1) * 64, :, :].astype(jnp.bfloat16)
    v_dma = pltpu.make_async_copy(
        v_ref.at[0, :, pl.ds(my * H_LOC, H_LOC), :], kvbuf, v_sem
    )
    v_dma.start()

    qp[:, :] = jnp.dot(
        xp[:, :], wq_ref[:, :].astype(jnp.bfloat16),
        preferred_element_type=jnp.float32,
    ).astype(jnp.bfloat16)

    v_dma.wait()
    for g in range(NG):
        for j in range(JPG):
            b = g + NG * j
            dst = slice(g * GROUP + j * 64, g * GROUP + (j + 1) * 64)
            vp[dst, :, :] = kvbuf[b * 64:(b + 1) * 64, :, :].astype(jnp.bfloat16)

    for g in range(NG):
        rows = slice(g * GROUP, (g + 1) * GROUP)
        for h in range(H_LOC):
            cols = slice(h * DH, (h + 1) * DH)
            s = lax.dot_general(
                qp[rows, cols], kp[rows, h, :],
                (((1,), (1,)), ((), ())),
                preferred_element_type=jnp.float32,
            ) * SCALE
            e = jnp.exp(s)
            denom = jnp.sum(e, axis=1, keepdims=True)
            ctx = jnp.dot(
                e.astype(jnp.bfloat16), vp[rows, h, :],
                preferred_element_type=jnp.float32,
            ) * (1.0 / denom)
            qp[rows, cols] = ctx.astype(jnp.bfloat16)

    wo_b = wo_ref[:, :].astype(jnp.bfloat16)
    offs = [base for base, _, _ in HD_INST]
    rs0 = []
    for i, (base, size, masks) in enumerate(HD_INST):
        for k, b in enumerate(range(base // 64, (base + size) // 64)):
            xp[k * 64:(k + 1) * 64, :] = (
                qp[(b % NG) * GROUP + (b // NG) * 64:
                   (b % NG) * GROUP + (b // NG) * 64 + 64, :]
            )
        p = jnp.dot(
            xp[0:size, :], wo_b, preferred_element_type=jnp.float32
        )
        out_ref[base:base + size, :] = p.astype(jnp.bfloat16)

        m = masks[0]
        half = size >> 1
        t = _tbit(my, m)
        rdma = pltpu.make_async_remote_copy(
            src_ref=out_ref.at[pl.ds(offs[i] + (1 - t) * half, half), :],
            dst_ref=bufs[i][0],
            send_sem=rs_send.at[i, 0],
            recv_sem=rs_recv.at[i, 0],
            device_id=(jnp.bitwise_xor(my, m),),
            device_id_type=pl.DeviceIdType.MESH,
        )
        rdma.start()
        rs0.append((rdma, i, half, t))

    for r in range(3):
        if r == 0:
            rdmas = rs0
        else:
            rdmas = []
            for i, (base, size, masks) in enumerate(HD_INST):
                m = masks[r]
                half = size >> (r + 1)
                t = _tbit(my, m)
                partner = jnp.bitwise_xor(my, m)
                send_off = offs[i] + (1 - t) * half
                rdma = pltpu.make_async_remote_copy(
                    src_ref=out_ref.at[pl.ds(send_off, half), :],
                    dst_ref=bufs[i][r],
                    send_sem=rs_send.at[i, r],
                    recv_sem=rs_recv.at[i, r],
                    device_id=(partner,),
                    device_id_type=pl.DeviceIdType.MESH,
                )
                rdma.start()
                rdmas.append((rdma, i, half, t))
        for rdma, i, half, t in rdmas:
            rdma.wait()
            keep_off = offs[i] + t * half
            reg = pl.ds(keep_off, half)
            out_ref[reg, :] = (
                out_ref[reg, :].astype(jnp.float32)
                + bufs[i][r][:, :].astype(jnp.float32)
            ).astype(jnp.bfloat16)
            offs[i] = keep_off


    for r in (2, 1, 0):
        rdmas = []
        for i, (base, size, masks) in enumerate(HD_INST):
            m = masks[r]
            rsz = size >> (r + 1)
            t = _tbit(my, m)
            partner = jnp.bitwise_xor(my, m)
            rdma = pltpu.make_async_remote_copy(
                src_ref=out_ref.at[pl.ds(offs[i], rsz), :],
                dst_ref=out_ref.at[pl.ds(offs[i], rsz), :],
                send_sem=ag_send.at[i, r],
                recv_sem=ag_recv.at[i, r],
                device_id=(partner,),
                device_id_type=pl.DeviceIdType.MESH,
            )
            rdma.start()
            rdmas.append((rdma, i, rsz, t))
        for rdma, i, rsz, t in rdmas:
            rdma.wait()
            offs[i] = offs[i] - t * rsz


def kernel(x, Wq, K_ext, V_ext, Wo):
    xb = x[0].astype(jnp.bfloat16)
    out2d = pl.pallas_call(
        _body,
        out_shape=jax.ShapeDtypeStruct((SQ, D_MODEL), jnp.bfloat16),
        in_specs=[
            pl.BlockSpec(memory_space=pltpu.VMEM),
            pl.BlockSpec(memory_space=pltpu.VMEM),
            pl.BlockSpec(memory_space=pltpu.MemorySpace.HBM),
            pl.BlockSpec(memory_space=pltpu.MemorySpace.HBM),
            pl.BlockSpec(memory_space=pltpu.VMEM),
        ],
        out_specs=pl.BlockSpec(memory_space=pltpu.VMEM),
        scratch_shapes=[
            pltpu.VMEM((SQ, D_MODEL), jnp.bfloat16),
            pltpu.VMEM((SQ, H_LOC, DH), jnp.bfloat16),
            pltpu.VMEM((SQ, H_LOC, DH), jnp.bfloat16),
            pltpu.VMEM((SQ, D_MODEL), jnp.bfloat16),
            pltpu.VMEM((SQ, H_LOC, DH), jnp.float32),
        ] + [
            pltpu.VMEM((size >> (r + 1), D_MODEL), jnp.bfloat16)
            for _, size, _ in HD_INST
            for r in range(3)
        ] + [
            pltpu.SemaphoreType.DMA((3, 3)),
            pltpu.SemaphoreType.DMA((3, 3)),
            pltpu.SemaphoreType.DMA((3, 3)),
            pltpu.SemaphoreType.DMA((3, 3)),
            pltpu.SemaphoreType.DMA,
            pltpu.SemaphoreType.DMA,
        ],
        compiler_params=pltpu.CompilerParams(
            vmem_limit_bytes=128 * 1024 * 1024,
        ),
    )(xb, Wq, K_ext, V_ext, Wo)
    return out2d.reshape(1, SQ, D_MODEL)


# device time: 102084 ns/iter; 1.0297x vs baseline; 1.0297x over previous
import jax
import jax.numpy as jnp
from jax import lax
from jax.experimental import pallas as pl
from jax.experimental.pallas import tpu as pltpu

N_DEV = 8
SQ = 2048
D_MODEL = 1024
H_LOC = 8
DH = 128
NG = 4
NB = SQ // 64
JPG = NB // NG
GROUP = JPG * 64
CHUNK = SQ // N_DEV
SCALE = 0.08838834764831843


HD_INST = (
    (0, 768, (1, 3, 4)),
    (768, 768, (3, 4, 1)),
    (1536, 512, (4, 1, 3)),
)


def _tbit(my, m):
    p0 = lax.rem(my, 2)
    p1 = lax.rem(my // 2, 2)
    p2 = my // 4
    if m == 1:
        return lax.rem(p0 + p1, 2)
    if m == 3:
        return p1
    return p2


def _body(x_ref, wq_ref, k_ref, v_ref, wo_ref, out_ref,
          xp, kp, vp, qp, kvbuf, *comm):
    bufs = [[comm[i * 3 + r] for r in range(3)] for i in range(3)]
    rs_send, rs_recv, ag_send, ag_recv = comm[9:13]
    k_sem, v_sem = comm[13], comm[14]
    my = lax.axis_index("i")

    k_dma = pltpu.make_async_copy(
        k_ref.at[0, :, pl.ds(my * H_LOC, H_LOC), :], kvbuf, k_sem
    )
    k_dma.start()

    for g in range(NG):
        for j in range(JPG):
            b = g + NG * j
            dst = slice(g * GROUP + j * 64, g * GROUP + (j + 1) * 64)
            xp[dst, :] = x_ref[b * 64:(b + 1) * 64, :]

    k_dma.wait()
    for g in range(NG):
        for j in range(JPG):
            b = g + NG * j
            dst = slice(g * GROUP + j * 64, g * GROUP + (j + 1) * 64)
            kp[dst, :] = jnp.reshape(
                kvbuf[b * 64:(b + 1) * 64, :, :], (64, H_LOC * DH)
            ).astype(jnp.bfloat16)
    v_dma = pltpu.make_async_copy(
        v_ref.at[0, :, pl.ds(my * H_LOC, H_LOC), :], kvbuf, v_sem
    )
    v_dma.start()

    qp[:, :] = jnp.dot(
        xp[:, :], wq_ref[:, :].astype(jnp.bfloat16),
        preferred_element_type=jnp.float32,
    ).astype(jnp.bfloat16)

    v_dma.wait()
    for g in range(NG):
        for j in range(JPG):
            b = g + NG * j
            dst = slice(g * GROUP + j * 64, g * GROUP + (j + 1) * 64)
            vp[dst, :] = jnp.reshape(
                kvbuf[b * 64:(b + 1) * 64, :, :], (64, H_LOC * DH)
            ).astype(jnp.bfloat16)

    def _att_step(idx, carry):
        g = idx // H_LOC
        h = lax.rem(idx, H_LOC)
        rows = pl.ds(g * GROUP, GROUP)
        s = lax.dot_general(
            qp[rows, pl.ds(h * DH, DH)], kp[rows, pl.ds(h * DH, DH)],
            (((1,), (1,)), ((), ())),
            preferred_element_type=jnp.float32,
        ) * SCALE
        e = jnp.exp(s)
        denom = jnp.sum(e, axis=1, keepdims=True)
        ctx = jnp.dot(
            e.astype(jnp.bfloat16), vp[rows, pl.ds(h * DH, DH)],
            preferred_element_type=jnp.float32,
        ) * (1.0 / denom)
        qp[rows, pl.ds(h * DH, DH)] = ctx.astype(jnp.bfloat16)
        return carry

    lax.fori_loop(0, NG * H_LOC, _att_step, 0)

    wo_b = wo_ref[:, :].astype(jnp.bfloat16)
    offs = [base for base, _, _ in HD_INST]
    rs0 = []
    for i, (base, size, masks) in enumerate(HD_INST):
        for k, b in enumerate(range(base // 64, (base + size) // 64)):
            xp[k * 64:(k + 1) * 64, :] = (
                qp[(b % NG) * GROUP + (b // NG) * 64:
                   (b % NG) * GROUP + (b // NG) * 64 + 64, :]
            )
        p = jnp.dot(
            xp[0:size, :], wo_b, preferred_element_type=jnp.float32
        )
        out_ref[base:base + size, :] = p.astype(jnp.bfloat16)

        m = masks[0]
        half = size >> 1
        t = _tbit(my, m)
        rdma = pltpu.make_async_remote_copy(
            src_ref=out_ref.at[pl.ds(offs[i] + (1 - t) * half, half), :],
            dst_ref=bufs[i][0],
            send_sem=rs_send.at[i, 0],
            recv_sem=rs_recv.at[i, 0],
            device_id=(jnp.bitwise_xor(my, m),),
            device_id_type=pl.DeviceIdType.MESH,
        )
        rdma.start()
        rs0.append((rdma, i, half, t))

    for r in range(3):
        if r == 0:
            rdmas = rs0
        else:
            rdmas = []
            for i, (base, size, masks) in enumerate(HD_INST):
                m = masks[r]
                half = size >> (r + 1)
                t = _tbit(my, m)
                partner = jnp.bitwise_xor(my, m)
                send_off = offs[i] + (1 - t) * half
                rdma = pltpu.make_async_remote_copy(
                    src_ref=out_ref.at[pl.ds(send_off, half), :],
                    dst_ref=bufs[i][r],
                    send_sem=rs_send.at[i, r],
                    recv_sem=rs_recv.at[i, r],
                    device_id=(partner,),
                    device_id_type=pl.DeviceIdType.MESH,
                )
                rdma.start()
                rdmas.append((rdma, i, half, t))
        for rdma, i, half, t in rdmas:
            rdma.wait()
            keep_off = offs[i] + t * half
            reg = pl.ds(keep_off, half)
            out_ref[reg, :] = (
                out_ref[reg, :].astype(jnp.float32)
                + bufs[i][r][:, :].astype(jnp.float32)
            ).astype(jnp.bfloat16)
            offs[i] = keep_off


    for r in (2, 1, 0):
        rdmas = []
        for i, (base, size, masks) in enumerate(HD_INST):
            m = masks[r]
            rsz = size >> (r + 1)
            t = _tbit(my, m)
            partner = jnp.bitwise_xor(my, m)
            rdma = pltpu.make_async_remote_copy(
                src_ref=out_ref.at[pl.ds(offs[i], rsz), :],
                dst_ref=out_ref.at[pl.ds(offs[i], rsz), :],
                send_sem=ag_send.at[i, r],
                recv_sem=ag_recv.at[i, r],
                device_id=(partner,),
                device_id_type=pl.DeviceIdType.MESH,
            )
            rdma.start()
            rdmas.append((rdma, i, rsz, t))
        for rdma, i, rsz, t in rdmas:
            rdma.wait()
            offs[i] = offs[i] - t * rsz


def kernel(x, Wq, K_ext, V_ext, Wo):
    xb = x[0].astype(jnp.bfloat16)
    out2d = pl.pallas_call(
        _body,
        out_shape=jax.ShapeDtypeStruct((SQ, D_MODEL), jnp.bfloat16),
        in_specs=[
            pl.BlockSpec(memory_space=pltpu.VMEM),
            pl.BlockSpec(memory_space=pltpu.VMEM),
            pl.BlockSpec(memory_space=pltpu.MemorySpace.HBM),
            pl.BlockSpec(memory_space=pltpu.MemorySpace.HBM),
            pl.BlockSpec(memory_space=pltpu.VMEM),
        ],
        out_specs=pl.BlockSpec(memory_space=pltpu.VMEM),
        scratch_shapes=[
            pltpu.VMEM((SQ, D_MODEL), jnp.bfloat16),
            pltpu.VMEM((SQ, H_LOC * DH), jnp.bfloat16),
            pltpu.VMEM((SQ, H_LOC * DH), jnp.bfloat16),
            pltpu.VMEM((SQ, D_MODEL), jnp.bfloat16),
            pltpu.VMEM((SQ, H_LOC, DH), jnp.float32),
        ] + [
            pltpu.VMEM((size >> (r + 1), D_MODEL), jnp.bfloat16)
            for _, size, _ in HD_INST
            for r in range(3)
        ] + [
            pltpu.SemaphoreType.DMA((3, 3)),
            pltpu.SemaphoreType.DMA((3, 3)),
            pltpu.SemaphoreType.DMA((3, 3)),
            pltpu.SemaphoreType.DMA((3, 3)),
            pltpu.SemaphoreType.DMA,
            pltpu.SemaphoreType.DMA,
        ],
        compiler_params=pltpu.CompilerParams(
            vmem_limit_bytes=128 * 1024 * 1024,
        ),
    )(xb, Wq, K_ext, V_ext, Wo)
    return out2d.reshape(1, SQ, D_MODEL)
